# Initial kernel scaffold; baseline (speedup 1.0000x reference)
import jax
import jax.numpy as jnp
from jax.experimental import pallas as pl


def kernel(x, edge_index, edge_attr, batch, W1, b1, W2, b2, lin_W, lin_b):
    raise NotImplementedError("write your pallas kernel here")


# speedup vs baseline: 2.1847x; 2.1847x over previous
import functools

import jax
import jax.numpy as jnp
from jax.experimental import pallas as pl


def _pool_head_body(sq_ref, batch_ref, o_ref):
    g = o_ref.shape[1]
    npad = sq_ref.shape[1]
    seg = jax.lax.broadcasted_iota(jnp.int32, (g, npad), 0)
    mask = seg == batch_ref[0][None, :]
    sums = jnp.sum(jnp.where(mask, sq_ref[0][None, :], 0.0), axis=1)
    cnt = jnp.sum(jnp.where(mask, 1.0, 0.0), axis=1)
    o_ref[0, :] = sums
    o_ref[1, :] = cnt


def _pool_head(sq, batch, g):
    n = sq.shape[0]
    npad = ((n + 127) // 128) * 128
    sq_p = jnp.zeros((1, npad), jnp.float32).at[0, :n].set(sq)
    batch_p = jnp.full((1, npad), -1, jnp.int32).at[0, :n].set(batch)
    out = pl.pallas_call(
        _pool_head_body,
        out_shape=jax.ShapeDtypeStruct((2, g), jnp.float32),
    )(sq_p, batch_p)
    return out[0], out[1]


def kernel(x, edge_index, edge_attr, batch, W1, b1, W2, b2, lin_W, lin_b):
    n = x.shape[0]
    g = 64
    src, dst = edge_index[0], edge_index[1]
    deg = jnp.zeros((n,), jnp.float32).at[dst].add(1.0) + 1.0
    dis = jax.lax.rsqrt(deg)
    h1 = x @ W1
    u1 = h1 * dis[:, None]
    agg1 = jnp.zeros_like(u1).at[dst].add(u1[src])
    out1 = jax.nn.relu(dis[:, None] * (agg1 + u1) + b1)
    w = W2 @ lin_W
    q = (out1 @ w)[:, 0]
    qu = dis * q
    aggq = jnp.zeros((n,), jnp.float32).at[dst].add(qu[src])
    sq = dis * (aggq + qu)
    sums, cnt = _pool_head(sq, batch, g)
    const = (b2 @ lin_W)[0] + lin_b[0]
    return jnp.where(cnt > 0, sums / jnp.maximum(cnt, 1.0) + const, lin_b[0])


# speedup vs baseline: 19.7016x; 9.0180x over previous
import functools

import jax
import jax.numpy as jnp
from jax import lax
from jax.experimental import pallas as pl
from jax.experimental.pallas import tpu as pltpu
from jax.experimental.pallas import tpu_sc as plsc

_NC = 2
_NS = 16
_CHUNK = 128
_G = 64


def _sc_mesh():
    return plsc.VectorSubcoreMesh(core_axis_name="c", subcore_axis_name="s")


def _make_sc_pass(n, e, d, gather):
    per_sc = e // _NC
    per_tile = per_sc // _NS
    nfull = per_tile // _CHUNK
    tail = per_tile % _CHUNK
    rpt = n // _NS

    scratch = [
        pltpu.VMEM((_CHUNK,), jnp.int32),
        pltpu.VMEM((max(tail, 1),), jnp.int32),
        pltpu.VMEM((_CHUNK, d), jnp.float32),
        pltpu.VMEM_SHARED((n, d), jnp.float32),
        pltpu.SemaphoreType.DMA,
    ]
    if gather:
        scratch = ([pltpu.VMEM((_CHUNK,), jnp.int32),
                    pltpu.VMEM((max(tail, 1),), jnp.int32),
                    pltpu.VMEM((max(tail, 1), d), jnp.float32)]
                   + scratch)

    out_type = jax.ShapeDtypeStruct((_NC * n, d), jnp.float32)
    params = pltpu.CompilerParams(use_tc_tiling_on_sc=False)

    if gather:
        @functools.partial(pl.kernel, out_type=out_type, mesh=_sc_mesh(),
                           scratch_types=scratch, compiler_params=params)
        def sc_pass(src_hbm, dst_hbm, vals_hbm, zeros_hbm, out_hbm,
                    src_v, srct_v, rowst_v, dst_v, dstt_v, rows_v, agg, sem):
            cid = lax.axis_index("c")
            sid = lax.axis_index("s")
            row0 = sid * rpt
            pltpu.sync_copy(zeros_hbm, agg.at[pl.ds(row0, rpt), :])
            plsc.subcore_barrier()
            ebase = cid * per_sc + sid * per_tile

            def chunk(j, carry):
                base = ebase + j * _CHUNK
                pltpu.sync_copy(src_hbm.at[pl.ds(base, _CHUNK)], src_v)
                pltpu.sync_copy(dst_hbm.at[pl.ds(base, _CHUNK)], dst_v)
                pltpu.async_copy(vals_hbm.at[src_v], rows_v, sem).wait()
                pltpu.sync_copy(rows_v, agg.at[dst_v], add=True)
                return carry

            lax.fori_loop(0, nfull, chunk, 0)
            if tail:
                base = ebase + nfull * _CHUNK
                pltpu.sync_copy(src_hbm.at[pl.ds(base, tail)], srct_v)
                pltpu.sync_copy(dst_hbm.at[pl.ds(base, tail)], dstt_v)
                pltpu.async_copy(vals_hbm.at[srct_v], rowst_v, sem).wait()
                pltpu.sync_copy(rowst_v, agg.at[dstt_v], add=True)
            plsc.subcore_barrier()
            pltpu.sync_copy(agg.at[pl.ds(row0, rpt), :],
                            out_hbm.at[pl.ds(cid * n + row0, rpt), :])
    else:
        @functools.partial(pl.kernel, out_type=out_type, mesh=_sc_mesh(),
                           scratch_types=scratch, compiler_params=params)
        def sc_pass(dst_hbm, ones_hbm, zeros_hbm, out_hbm,
                    dst_v, dstt_v, rows_v, agg, sem):
            cid = lax.axis_index("c")
            sid = lax.axis_index("s")
            row0 = sid * rpt
            pltpu.sync_copy(ones_hbm, rows_v)
            pltpu.sync_copy(zeros_hbm, agg.at[pl.ds(row0, rpt), :])
            plsc.subcore_barrier()
            ebase = cid * per_sc + sid * per_tile

            def chunk(j, carry):
                base = ebase + j * _CHUNK
                pltpu.sync_copy(dst_hbm.at[pl.ds(base, _CHUNK)], dst_v)
                pltpu.sync_copy(rows_v, agg.at[dst_v], add=True)
                return carry

            lax.fori_loop(0, nfull, chunk, 0)
            if tail:
                base = ebase + nfull * _CHUNK
                pltpu.sync_copy(dst_hbm.at[pl.ds(base, tail)], dstt_v)
                pltpu.sync_copy(rows_v.at[pl.ds(0, tail), :], agg.at[dstt_v],
                                add=True)
            plsc.subcore_barrier()
            pltpu.sync_copy(agg.at[pl.ds(row0, rpt), :],
                            out_hbm.at[pl.ds(cid * n + row0, rpt), :])

    return sc_pass


def _tc_a_body(x_ref, w1_ref, d0_ref, d1_ref, o_ref):
    h = jnp.dot(x_ref[...], w1_ref[...], preferred_element_type=jnp.float32)
    deg = d0_ref[:, 0:1] + d1_ref[:, 0:1] + 1.0
    o_ref[...] = h * lax.rsqrt(deg)


def _tc_b_body(a0_ref, a1_ref, u1_ref, d0_ref, d1_ref, b1_ref, w2_ref,
               linw_ref, o_ref):
    deg = d0_ref[:, 0:1] + d1_ref[:, 0:1] + 1.0
    dis = lax.rsqrt(deg)
    out1 = jax.nn.relu(dis * (a0_ref[...] + a1_ref[...] + u1_ref[...])
                       + b1_ref[...])
    w = jnp.dot(w2_ref[...], linw_ref[...], preferred_element_type=jnp.float32)
    q = jnp.dot(out1, w, preferred_element_type=jnp.float32)
    o_ref[...] = jnp.broadcast_to(dis * q, o_ref.shape)


def _tc_c_body(a0_ref, a1_ref, qu_ref, d0_ref, d1_ref, batch_ref, b2_ref,
               linw_ref, linb_ref, o_ref):
    n = a0_ref.shape[0]
    deg = d0_ref[:, 0:1] + d1_ref[:, 0:1] + 1.0
    dis = lax.rsqrt(deg)
    sq = dis * (a0_ref[:, 0:1] + a1_ref[:, 0:1] + qu_ref[:, 0:1])
    segid = lax.broadcasted_iota(jnp.int32, (n, _G), 1)
    mask = jnp.where(segid == batch_ref[...], 1.0, 0.0)
    sums = jnp.sum(mask * sq, axis=0)
    cnt = jnp.sum(mask, axis=0)
    const = jnp.dot(b2_ref[...], linw_ref[...],
                    preferred_element_type=jnp.float32)[0, 0] + linb_ref[0, 0]
    o_ref[0, :] = jnp.where(cnt > 0, sums / jnp.maximum(cnt, 1.0) + const,
                            linb_ref[0, 0])


def kernel(x, edge_index, edge_attr, batch, W1, b1, W2, b2, lin_W, lin_b):
    n = x.shape[0]
    e = edge_index.shape[1]
    din = x.shape[1]
    h = W1.shape[1]
    src = edge_index[0]
    dst = edge_index[1]
    rpt = -(-n // (8 * _NS)) * 8
    npad = rpt * _NS

    ones8 = jnp.ones((_CHUNK, 8), jnp.float32)
    zeros8 = jnp.zeros((rpt, 8), jnp.float32)
    zerosh = jnp.zeros((rpt, h), jnp.float32)

    degp = _make_sc_pass(npad, e, 8, gather=False)(dst, ones8, zeros8)
    d0, d1 = degp[:n], degp[npad:npad + n]

    rb = 2000
    grid = (n // rb,)
    u1 = pl.pallas_call(
        _tc_a_body,
        grid=grid,
        in_specs=[
            pl.BlockSpec((rb, din), lambda i: (i, 0)),
            pl.BlockSpec((din, h), lambda i: (0, 0)),
            pl.BlockSpec((rb, 8), lambda i: (i, 0)),
            pl.BlockSpec((rb, 8), lambda i: (i, 0)),
        ],
        out_specs=pl.BlockSpec((rb, h), lambda i: (i, 0)),
        out_shape=jax.ShapeDtypeStruct((n, h), jnp.float32),
    )(x, W1, d0, d1)

    aggp = _make_sc_pass(npad, e, h, gather=True)(src, dst, u1, zerosh)

    qu = pl.pallas_call(
        _tc_b_body,
        grid=grid,
        in_specs=[
            pl.BlockSpec((rb, h), lambda i: (i, 0)),
            pl.BlockSpec((rb, h), lambda i: (i, 0)),
            pl.BlockSpec((rb, h), lambda i: (i, 0)),
            pl.BlockSpec((rb, 8), lambda i: (i, 0)),
            pl.BlockSpec((rb, 8), lambda i: (i, 0)),
            pl.BlockSpec((1, h), lambda i: (0, 0)),
            pl.BlockSpec((h, h), lambda i: (0, 0)),
            pl.BlockSpec((h, 1), lambda i: (0, 0)),
        ],
        out_specs=pl.BlockSpec((rb, 8), lambda i: (i, 0)),
        out_shape=jax.ShapeDtypeStruct((n, 8), jnp.float32),
    )(aggp[:n], aggp[npad:npad + n], u1, d0, d1, b1.reshape(1, h), W2, lin_W)

    aggqp = _make_sc_pass(npad, e, 8, gather=True)(src, dst, qu, zeros8)

    out = pl.pallas_call(
        _tc_c_body,
        in_specs=[
            pl.BlockSpec((n, 8), lambda: (0, 0)),
            pl.BlockSpec((n, 8), lambda: (0, 0)),
            pl.BlockSpec((n, 8), lambda: (0, 0)),
            pl.BlockSpec((n, 8), lambda: (0, 0)),
            pl.BlockSpec((n, 8), lambda: (0, 0)),
            pl.BlockSpec((n, 1), lambda: (0, 0)),
            pl.BlockSpec((1, h), lambda: (0, 0)),
            pl.BlockSpec((h, 1), lambda: (0, 0)),
            pl.BlockSpec((1, 1), lambda: (0, 0)),
        ],
        out_specs=pl.BlockSpec((1, _G), lambda: (0, 0)),
        out_shape=jax.ShapeDtypeStruct((1, _G), jnp.float32),
    )(aggqp[:n], aggqp[npad:npad + n], qu, d0, d1, batch.reshape(n, 1),
      b2.reshape(1, h), lin_W, lin_b.reshape(1, 1))
    return out[0]


# speedup vs baseline: 45.9011x; 2.3298x over previous
import functools

import jax
import jax.numpy as jnp
from jax import lax
from jax.experimental import pallas as pl
from jax.experimental.pallas import tpu as pltpu
from jax.experimental.pallas import tpu_sc as plsc

_NC = 2
_NS = 16
_CHUNK = 128
_G = 64


def _sc_mesh():
    return plsc.VectorSubcoreMesh(core_axis_name="c", subcore_axis_name="s")


def _make_sc_pass(n, e, d, gather):
    erows = e // _CHUNK
    per_sc = erows // _NC
    q = per_sc // _NS
    r = per_sc % _NS
    nb = next(b for b in (6, 5, 4, 3, 2, 1) if q % b == 0)
    ng = q // nb
    rpt = n // _NS

    scratch = []
    if gather:
        scratch += [pltpu.VMEM((q, _CHUNK), jnp.int32),
                    pltpu.VMEM((1, _CHUNK), jnp.int32)]
    scratch += [
        pltpu.VMEM((q, _CHUNK), jnp.int32),
        pltpu.VMEM((1, _CHUNK), jnp.int32),
        pltpu.VMEM(((nb if gather else 1), _CHUNK, d), jnp.float32),
        pltpu.VMEM_SHARED((n, d), jnp.float32),
        pltpu.SemaphoreType.DMA,
    ]
    scratch += [pltpu.SemaphoreType.DMA] * (2 * nb if gather else nb)

    out_type = jax.ShapeDtypeStruct((_NC * n, d), jnp.float32)
    params = pltpu.CompilerParams(use_tc_tiling_on_sc=False)

    if gather:
        @functools.partial(pl.kernel, out_type=out_type, mesh=_sc_mesh(),
                           scratch_types=scratch, compiler_params=params)
        def sc_pass(srcR, dstR, vals, zeros, out_hbm,
                    srcb, srcx, dstb, dstx, rows, agg, isem, *sems):
            gsem, ssem = sems[:nb], sems[nb:]
            cid = lax.axis_index("c")
            sid = lax.axis_index("s")
            row0 = sid * rpt
            er0 = cid * per_sc + sid * q
            xrow = cid * per_sc + _NS * q + sid
            pltpu.async_copy(srcR.at[pl.ds(er0, q), :], srcb, isem)
            pltpu.async_copy(dstR.at[pl.ds(er0, q), :], dstb, isem)

            @pl.when(sid < r)
            def _():
                pltpu.async_copy(srcR.at[pl.ds(xrow, 1), :], srcx, isem)
                pltpu.async_copy(dstR.at[pl.ds(xrow, 1), :], dstx, isem)

            pltpu.sync_copy(zeros, agg.at[pl.ds(row0, rpt), :])
            pltpu.make_async_copy(srcR.at[pl.ds(er0, q), :], srcb, isem).wait()
            pltpu.make_async_copy(dstR.at[pl.ds(er0, q), :], dstb, isem).wait()

            @pl.when(sid < r)
            def _():
                pltpu.make_async_copy(srcR.at[pl.ds(xrow, 1), :], srcx,
                                      isem).wait()
                pltpu.make_async_copy(dstR.at[pl.ds(xrow, 1), :], dstx,
                                      isem).wait()

            for u in range(nb):
                pltpu.async_copy(vals.at[srcb.at[u]], rows.at[u], gsem[u])
            plsc.subcore_barrier()

            def group(jj, carry):
                for u in range(nb):
                    pltpu.make_async_copy(vals.at[srcb.at[jj * nb + u]],
                                          rows.at[u], gsem[u]).wait()
                    pltpu.async_copy(rows.at[u], agg.at[dstb.at[jj * nb + u]],
                                     ssem[u], add=True)

                @pl.when(jj < ng - 1)
                def _():
                    for u in range(nb):
                        pltpu.make_async_copy(
                            rows.at[u], agg.at[dstb.at[jj * nb + u]],
                            ssem[u]).wait()
                        pltpu.async_copy(vals.at[srcb.at[(jj + 1) * nb + u]],
                                         rows.at[u], gsem[u])
                return carry

            lax.fori_loop(0, ng, group, 0)
            for u in range(nb):
                pltpu.make_async_copy(rows.at[u],
                                      agg.at[dstb.at[(ng - 1) * nb + u]],
                                      ssem[u]).wait()

            @pl.when(sid < r)
            def _():
                pltpu.async_copy(vals.at[srcx.at[0]], rows.at[0],
                                 gsem[0]).wait()
                pltpu.sync_copy(rows.at[0], agg.at[dstx.at[0]], add=True)

            plsc.subcore_barrier()
            pltpu.sync_copy(agg.at[pl.ds(row0, rpt), :],
                            out_hbm.at[pl.ds(cid * n + row0, rpt), :])
    else:
        @functools.partial(pl.kernel, out_type=out_type, mesh=_sc_mesh(),
                           scratch_types=scratch, compiler_params=params)
        def sc_pass(dstR, ones_hbm, zeros_hbm, out_hbm,
                    dstb, dstx, rows, agg, isem, *ssem):
            cid = lax.axis_index("c")
            sid = lax.axis_index("s")
            row0 = sid * rpt
            er0 = cid * per_sc + sid * q
            xrow = cid * per_sc + _NS * q + sid
            pltpu.async_copy(dstR.at[pl.ds(er0, q), :], dstb, isem)

            @pl.when(sid < r)
            def _():
                pltpu.async_copy(dstR.at[pl.ds(xrow, 1), :], dstx, isem)

            pltpu.sync_copy(ones_hbm, rows.at[0])
            pltpu.sync_copy(zeros_hbm, agg.at[pl.ds(row0, rpt), :])
            pltpu.make_async_copy(dstR.at[pl.ds(er0, q), :], dstb, isem).wait()

            @pl.when(sid < r)
            def _():
                pltpu.make_async_copy(dstR.at[pl.ds(xrow, 1), :], dstx,
                                      isem).wait()

            plsc.subcore_barrier()

            def group(jj, carry):
                @pl.when(jj > 0)
                def _():
                    for u in range(nb):
                        pltpu.make_async_copy(
                            rows.at[0], agg.at[dstb.at[(jj - 1) * nb + u]],
                            ssem[u]).wait()
                for u in range(nb):
                    pltpu.async_copy(rows.at[0], agg.at[dstb.at[jj * nb + u]],
                                     ssem[u], add=True)
                return carry

            lax.fori_loop(0, ng, group, 0)
            for u in range(nb):
                pltpu.make_async_copy(rows.at[0],
                                      agg.at[dstb.at[(ng - 1) * nb + u]],
                                      ssem[u]).wait()

            @pl.when(sid < r)
            def _():
                pltpu.sync_copy(rows.at[0], agg.at[dstx.at[0]], add=True)

            plsc.subcore_barrier()
            pltpu.sync_copy(agg.at[pl.ds(row0, rpt), :],
                            out_hbm.at[pl.ds(cid * n + row0, rpt), :])

    return sc_pass


def _tc_a_body(x_ref, w1_ref, d0_ref, d1_ref, o_ref):
    h = jnp.dot(x_ref[...], w1_ref[...], preferred_element_type=jnp.float32)
    deg = d0_ref[:, 0:1] + d1_ref[:, 0:1] + 1.0
    o_ref[...] = h * lax.rsqrt(deg)


def _tc_b_body(a0_ref, a1_ref, u1_ref, d0_ref, d1_ref, b1_ref, w2_ref,
               linw_ref, o_ref):
    deg = d0_ref[:, 0:1] + d1_ref[:, 0:1] + 1.0
    dis = lax.rsqrt(deg)
    out1 = jax.nn.relu(dis * (a0_ref[...] + a1_ref[...] + u1_ref[...])
                       + b1_ref[...])
    w = jnp.dot(w2_ref[...], linw_ref[...], preferred_element_type=jnp.float32)
    q = jnp.dot(out1, w, preferred_element_type=jnp.float32)
    o_ref[...] = jnp.broadcast_to(dis * q, o_ref.shape)


def _tc_c_body(a0_ref, a1_ref, qu_ref, d0_ref, d1_ref, batch_ref, b2_ref,
               linw_ref, linb_ref, o_ref):
    n = a0_ref.shape[0]
    deg = d0_ref[:, 0:1] + d1_ref[:, 0:1] + 1.0
    dis = lax.rsqrt(deg)
    sq = dis * (a0_ref[:, 0:1] + a1_ref[:, 0:1] + qu_ref[:, 0:1])
    segid = lax.broadcasted_iota(jnp.int32, (n, _G), 1)
    mask = jnp.where(segid == batch_ref[...], 1.0, 0.0)
    sums = jnp.sum(mask * sq, axis=0)
    cnt = jnp.sum(mask, axis=0)
    const = jnp.dot(b2_ref[...], linw_ref[...],
                    preferred_element_type=jnp.float32)[0, 0] + linb_ref[0, 0]
    o_ref[0, :] = jnp.where(cnt > 0, sums / jnp.maximum(cnt, 1.0) + const,
                            linb_ref[0, 0])


def kernel(x, edge_index, edge_attr, batch, W1, b1, W2, b2, lin_W, lin_b):
    n = x.shape[0]
    e = edge_index.shape[1]
    din = x.shape[1]
    h = W1.shape[1]
    srcR = edge_index[0].reshape(e // _CHUNK, _CHUNK)
    dstR = edge_index[1].reshape(e // _CHUNK, _CHUNK)
    rpt = -(-n // (8 * _NS)) * 8
    npad = rpt * _NS

    ones8 = jnp.ones((_CHUNK, 8), jnp.float32)
    zeros8 = jnp.zeros((rpt, 8), jnp.float32)
    zerosh = jnp.zeros((rpt, h), jnp.float32)

    degp = _make_sc_pass(npad, e, 8, gather=False)(dstR, ones8, zeros8)
    d0, d1 = degp[:n], degp[npad:npad + n]

    rb = 2000
    grid = (n // rb,)
    u1 = pl.pallas_call(
        _tc_a_body,
        grid=grid,
        in_specs=[
            pl.BlockSpec((rb, din), lambda i: (i, 0)),
            pl.BlockSpec((din, h), lambda i: (0, 0)),
            pl.BlockSpec((rb, 8), lambda i: (i, 0)),
            pl.BlockSpec((rb, 8), lambda i: (i, 0)),
        ],
        out_specs=pl.BlockSpec((rb, h), lambda i: (i, 0)),
        out_shape=jax.ShapeDtypeStruct((n, h), jnp.float32),
    )(x, W1, d0, d1)

    aggp = _make_sc_pass(npad, e, h, gather=True)(srcR, dstR, u1, zerosh)

    qu = pl.pallas_call(
        _tc_b_body,
        grid=grid,
        in_specs=[
            pl.BlockSpec((rb, h), lambda i: (i, 0)),
            pl.BlockSpec((rb, h), lambda i: (i, 0)),
            pl.BlockSpec((rb, h), lambda i: (i, 0)),
            pl.BlockSpec((rb, 8), lambda i: (i, 0)),
            pl.BlockSpec((rb, 8), lambda i: (i, 0)),
            pl.BlockSpec((1, h), lambda i: (0, 0)),
            pl.BlockSpec((h, h), lambda i: (0, 0)),
            pl.BlockSpec((h, 1), lambda i: (0, 0)),
        ],
        out_specs=pl.BlockSpec((rb, 8), lambda i: (i, 0)),
        out_shape=jax.ShapeDtypeStruct((n, 8), jnp.float32),
    )(aggp[:n], aggp[npad:npad + n], u1, d0, d1, b1.reshape(1, h), W2, lin_W)

    aggqp = _make_sc_pass(npad, e, 8, gather=True)(srcR, dstR, qu, zeros8)

    out = pl.pallas_call(
        _tc_c_body,
        in_specs=[
            pl.BlockSpec((n, 8), lambda: (0, 0)),
            pl.BlockSpec((n, 8), lambda: (0, 0)),
            pl.BlockSpec((n, 8), lambda: (0, 0)),
            pl.BlockSpec((n, 8), lambda: (0, 0)),
            pl.BlockSpec((n, 8), lambda: (0, 0)),
            pl.BlockSpec((n, 1), lambda: (0, 0)),
            pl.BlockSpec((1, h), lambda: (0, 0)),
            pl.BlockSpec((h, 1), lambda: (0, 0)),
            pl.BlockSpec((1, 1), lambda: (0, 0)),
        ],
        out_specs=pl.BlockSpec((1, _G), lambda: (0, 0)),
        out_shape=jax.ShapeDtypeStruct((1, _G), jnp.float32),
    )(aggqp[:n], aggqp[npad:npad + n], qu, d0, d1, batch.reshape(n, 1),
      b2.reshape(1, h), lin_W, lin_b.reshape(1, 1))
    return out[0]


# speedup vs baseline: 52.5374x; 1.1446x over previous
import functools

import jax
import jax.numpy as jnp
from jax import lax
from jax.experimental import pallas as pl
from jax.experimental.pallas import tpu as pltpu
from jax.experimental.pallas import tpu_sc as plsc

_NC = 2
_NS = 16
_CHUNK = 128
_G = 64


def _sc_mesh():
    return plsc.VectorSubcoreMesh(core_axis_name="c", subcore_axis_name="s")


def _make_sc_pass(n, e, d, gather):
    erows = e // _CHUNK
    per_sc = erows // _NC
    q = per_sc // _NS
    r = per_sc % _NS
    nb = next(b for b in (6, 5, 4, 3, 2, 1) if q % b == 0)
    ng = q // nb
    rpt = n // _NS

    scratch = []
    if gather:
        scratch += [pltpu.VMEM((q, _CHUNK), jnp.int32),
                    pltpu.VMEM((1, _CHUNK), jnp.int32)]
    scratch += [
        pltpu.VMEM((q, _CHUNK), jnp.int32),
        pltpu.VMEM((1, _CHUNK), jnp.int32),
        pltpu.VMEM(((nb if gather else 1), _CHUNK, d), jnp.float32),
        pltpu.VMEM_SHARED((n, d), jnp.float32),
        pltpu.SemaphoreType.DMA,
    ]
    scratch += [pltpu.SemaphoreType.DMA] * (2 * nb if gather else nb)

    out_type = jax.ShapeDtypeStruct((_NC * n, d), jnp.float32)
    params = pltpu.CompilerParams(use_tc_tiling_on_sc=False)

    if gather:
        @functools.partial(pl.kernel, out_type=out_type, mesh=_sc_mesh(),
                           scratch_types=scratch, compiler_params=params)
        def sc_pass(srcR, dstR, vals, zeros, out_hbm,
                    srcb, srcx, dstb, dstx, rows, agg, isem, *sems):
            gsem, ssem = sems[:nb], sems[nb:]
            cid = lax.axis_index("c")
            sid = lax.axis_index("s")
            row0 = sid * rpt
            er0 = cid * per_sc + sid * q
            xrow = cid * per_sc + _NS * q + sid
            pltpu.async_copy(srcR.at[pl.ds(er0, q), :], srcb, isem)
            pltpu.async_copy(dstR.at[pl.ds(er0, q), :], dstb, isem)

            @pl.when(sid < r)
            def _():
                pltpu.async_copy(srcR.at[pl.ds(xrow, 1), :], srcx, isem)
                pltpu.async_copy(dstR.at[pl.ds(xrow, 1), :], dstx, isem)

            pltpu.sync_copy(zeros, agg.at[pl.ds(row0, rpt), :])
            pltpu.make_async_copy(srcR.at[pl.ds(er0, q), :], srcb, isem).wait()
            pltpu.make_async_copy(dstR.at[pl.ds(er0, q), :], dstb, isem).wait()

            @pl.when(sid < r)
            def _():
                pltpu.make_async_copy(srcR.at[pl.ds(xrow, 1), :], srcx,
                                      isem).wait()
                pltpu.make_async_copy(dstR.at[pl.ds(xrow, 1), :], dstx,
                                      isem).wait()

            for u in range(nb):
                pltpu.async_copy(vals.at[srcb.at[u]], rows.at[u], gsem[u])
            plsc.subcore_barrier()

            def group(jj, carry):
                for u in range(nb):
                    pltpu.make_async_copy(vals.at[srcb.at[jj * nb + u]],
                                          rows.at[u], gsem[u]).wait()
                    pltpu.async_copy(rows.at[u], agg.at[dstb.at[jj * nb + u]],
                                     ssem[u], add=True)

                @pl.when(jj < ng - 1)
                def _():
                    for u in range(nb):
                        pltpu.make_async_copy(
                            rows.at[u], agg.at[dstb.at[jj * nb + u]],
                            ssem[u]).wait()
                        pltpu.async_copy(vals.at[srcb.at[(jj + 1) * nb + u]],
                                         rows.at[u], gsem[u])
                return carry

            lax.fori_loop(0, ng, group, 0)
            for u in range(nb):
                pltpu.make_async_copy(rows.at[u],
                                      agg.at[dstb.at[(ng - 1) * nb + u]],
                                      ssem[u]).wait()

            @pl.when(sid < r)
            def _():
                pltpu.async_copy(vals.at[srcx.at[0]], rows.at[0],
                                 gsem[0]).wait()
                pltpu.sync_copy(rows.at[0], agg.at[dstx.at[0]], add=True)

            plsc.subcore_barrier()
            pltpu.sync_copy(agg.at[pl.ds(row0, rpt), :],
                            out_hbm.at[pl.ds(cid * n + row0, rpt), :])
    else:
        @functools.partial(pl.kernel, out_type=out_type, mesh=_sc_mesh(),
                           scratch_types=scratch, compiler_params=params)
        def sc_pass(dstR, ones_hbm, zeros_hbm, out_hbm,
                    dstb, dstx, rows, agg, isem, *ssem):
            cid = lax.axis_index("c")
            sid = lax.axis_index("s")
            row0 = sid * rpt
            er0 = cid * per_sc + sid * q
            xrow = cid * per_sc + _NS * q + sid
            pltpu.async_copy(dstR.at[pl.ds(er0, q), :], dstb, isem)

            @pl.when(sid < r)
            def _():
                pltpu.async_copy(dstR.at[pl.ds(xrow, 1), :], dstx, isem)

            pltpu.sync_copy(ones_hbm, rows.at[0])
            pltpu.sync_copy(zeros_hbm, agg.at[pl.ds(row0, rpt), :])
            pltpu.make_async_copy(dstR.at[pl.ds(er0, q), :], dstb, isem).wait()

            @pl.when(sid < r)
            def _():
                pltpu.make_async_copy(dstR.at[pl.ds(xrow, 1), :], dstx,
                                      isem).wait()

            plsc.subcore_barrier()

            def group(jj, carry):
                @pl.when(jj > 0)
                def _():
                    for u in range(nb):
                        pltpu.make_async_copy(
                            rows.at[0], agg.at[dstb.at[(jj - 1) * nb + u]],
                            ssem[u]).wait()
                for u in range(nb):
                    pltpu.async_copy(rows.at[0], agg.at[dstb.at[jj * nb + u]],
                                     ssem[u], add=True)
                return carry

            lax.fori_loop(0, ng, group, 0)
            for u in range(nb):
                pltpu.make_async_copy(rows.at[0],
                                      agg.at[dstb.at[(ng - 1) * nb + u]],
                                      ssem[u]).wait()

            @pl.when(sid < r)
            def _():
                pltpu.sync_copy(rows.at[0], agg.at[dstx.at[0]], add=True)

            plsc.subcore_barrier()
            pltpu.sync_copy(agg.at[pl.ds(row0, rpt), :],
                            out_hbm.at[pl.ds(cid * n + row0, rpt), :])

    return sc_pass


def _make_sc_pool(n, e):
    erows = e // _CHUNK
    per_sc = erows // _NC
    q = per_sc // _NS
    r = per_sc % _NS
    npw = -(-n // (_NC * _NS * 16)) * 16
    nchunks = npw // 16

    scratch = [
        pltpu.VMEM((q, _CHUNK), jnp.int32),
        pltpu.VMEM((q, _CHUNK), jnp.int32),
        pltpu.VMEM((1, _CHUNK), jnp.int32),
        pltpu.VMEM((1, _CHUNK), jnp.int32),
        pltpu.VMEM((n,), jnp.float32),
        pltpu.VMEM((n,), jnp.float32),
        pltpu.VMEM((n,), jnp.int32),
        pltpu.VMEM((16, _G), jnp.float32),
        pltpu.VMEM((16, _G), jnp.float32),
        pltpu.SemaphoreType.DMA,
    ]

    @functools.partial(
        pl.kernel,
        out_type=jax.ShapeDtypeStruct((2 * _NC * _NS * 16, _G), jnp.float32),
        mesh=_sc_mesh(),
        scratch_types=scratch,
        compiler_params=pltpu.CompilerParams(use_tc_tiling_on_sc=False,
                                             needs_layout_passes=False),
    )
    def sc_pool(srcR, dstR, qu_hbm, dis_hbm, batch_hbm, zeros_hbm, out_hbm,
                srcb, dstb, srcx, dstx, qu_v, dis_v, bat_v, bk, ck, isem):
        cid = lax.axis_index("c")
        sid = lax.axis_index("s")
        wid = cid * _NS + sid
        er0 = cid * per_sc + sid * q
        xrow = cid * per_sc + _NS * q + sid
        pltpu.async_copy(srcR.at[pl.ds(er0, q), :], srcb, isem)
        pltpu.async_copy(dstR.at[pl.ds(er0, q), :], dstb, isem)
        pltpu.async_copy(qu_hbm, qu_v, isem)
        pltpu.async_copy(dis_hbm, dis_v, isem)
        pltpu.async_copy(batch_hbm, bat_v, isem)

        @pl.when(sid < r)
        def _():
            pltpu.async_copy(srcR.at[pl.ds(xrow, 1), :], srcx, isem)
            pltpu.async_copy(dstR.at[pl.ds(xrow, 1), :], dstx, isem)

        pltpu.sync_copy(zeros_hbm, bk)
        pltpu.sync_copy(zeros_hbm, ck)
        pltpu.make_async_copy(srcR.at[pl.ds(er0, q), :], srcb, isem).wait()
        pltpu.make_async_copy(dstR.at[pl.ds(er0, q), :], dstb, isem).wait()
        pltpu.make_async_copy(qu_hbm, qu_v, isem).wait()
        pltpu.make_async_copy(dis_hbm, dis_v, isem).wait()
        pltpu.make_async_copy(batch_hbm, bat_v, isem).wait()

        @pl.when(sid < r)
        def _():
            pltpu.make_async_copy(srcR.at[pl.ds(xrow, 1), :], srcx,
                                  isem).wait()
            pltpu.make_async_copy(dstR.at[pl.ds(xrow, 1), :], dstx,
                                  isem).wait()

        lane = jax.lax.broadcasted_iota(jnp.int32, (16,), 0)

        def edge_row(ref, er):
            for k in range(_CHUNK // 16):
                s16 = ref[0][er, pl.ds(16 * k, 16)]
                d16 = ref[1][er, pl.ds(16 * k, 16)]
                sv = plsc.load_gather(qu_v, [s16])
                dv = plsc.load_gather(dis_v, [d16])
                bv = plsc.load_gather(bat_v, [d16])
                plsc.addupdate_scatter(bk, [lane, bv], sv * dv)

        def erow_loop(er, carry):
            edge_row((srcb, dstb), er)
            return carry

        lax.fori_loop(0, q, erow_loop, 0)

        @pl.when(sid < r)
        def _():
            edge_row((srcx, dstx), 0)

        node0 = wid * npw
        ones16 = jnp.ones((16,), jnp.float32)

        def node_chunk(j, carry):
            idx = node0 + j * 16 + lane
            m = idx < n
            idxc = jnp.minimum(idx, n - 1)
            sv = plsc.load_gather(qu_v, [idxc])
            dv = plsc.load_gather(dis_v, [idxc])
            bv = plsc.load_gather(bat_v, [idxc])
            plsc.addupdate_scatter(bk, [lane, bv], sv * dv, mask=m)
            plsc.addupdate_scatter(ck, [lane, bv], ones16, mask=m)
            return carry

        lax.fori_loop(0, nchunks, node_chunk, 0)

        pltpu.sync_copy(bk, out_hbm.at[pl.ds(wid * 16, 16), :])
        pltpu.sync_copy(ck, out_hbm.at[pl.ds((_NC * _NS + wid) * 16, 16), :])

    return sc_pool


def _tc_a_body(x_ref, w1_ref, d0_ref, d1_ref, o_ref):
    h = jnp.dot(x_ref[...], w1_ref[...], preferred_element_type=jnp.float32)
    deg = d0_ref[:, 0:1] + d1_ref[:, 0:1] + 1.0
    o_ref[...] = h * lax.rsqrt(deg)


def _tc_b_body(a0_ref, a1_ref, u1_ref, d0_ref, d1_ref, b1_ref, w2_ref,
               linw_ref, b2_ref, linb_ref, qu_ref, dis_ref, c_ref):
    deg = d0_ref[:, 0:1] + d1_ref[:, 0:1] + 1.0
    dis = lax.rsqrt(deg)
    out1 = jax.nn.relu(dis * (a0_ref[...] + a1_ref[...] + u1_ref[...])
                       + b1_ref[...])
    w = jnp.dot(w2_ref[...], linw_ref[...], preferred_element_type=jnp.float32)
    q = jnp.dot(out1, w, preferred_element_type=jnp.float32)
    qu_ref[...] = dis * q
    dis_ref[...] = dis
    c_ref[...] = jnp.dot(b2_ref[...], linw_ref[...],
                         preferred_element_type=jnp.float32) + linb_ref[...]


def kernel(x, edge_index, edge_attr, batch, W1, b1, W2, b2, lin_W, lin_b):
    n = x.shape[0]
    e = edge_index.shape[1]
    din = x.shape[1]
    h = W1.shape[1]
    srcR = edge_index[0].reshape(e // _CHUNK, _CHUNK)
    dstR = edge_index[1].reshape(e // _CHUNK, _CHUNK)
    rpt = -(-n // (8 * _NS)) * 8
    npad = rpt * _NS

    ones8 = jnp.ones((_CHUNK, 8), jnp.float32)
    zeros8 = jnp.zeros((rpt, 8), jnp.float32)
    zerosh = jnp.zeros((rpt, h), jnp.float32)

    degp = _make_sc_pass(npad, e, 8, gather=False)(dstR, ones8, zeros8)
    d0, d1 = degp[:n], degp[npad:npad + n]

    rb = 2000
    grid = (n // rb,)
    u1 = pl.pallas_call(
        _tc_a_body,
        grid=grid,
        in_specs=[
            pl.BlockSpec((rb, din), lambda i: (i, 0)),
            pl.BlockSpec((din, h), lambda i: (0, 0)),
            pl.BlockSpec((rb, 8), lambda i: (i, 0)),
            pl.BlockSpec((rb, 8), lambda i: (i, 0)),
        ],
        out_specs=pl.BlockSpec((rb, h), lambda i: (i, 0)),
        out_shape=jax.ShapeDtypeStruct((n, h), jnp.float32),
    )(x, W1, d0, d1)

    aggp = _make_sc_pass(npad, e, h, gather=True)(srcR, dstR, u1, zerosh)

    qu, dis, const = pl.pallas_call(
        _tc_b_body,
        grid=grid,
        in_specs=[
            pl.BlockSpec((rb, h), lambda i: (i, 0)),
            pl.BlockSpec((rb, h), lambda i: (i, 0)),
            pl.BlockSpec((rb, h), lambda i: (i, 0)),
            pl.BlockSpec((rb, 8), lambda i: (i, 0)),
            pl.BlockSpec((rb, 8), lambda i: (i, 0)),
            pl.BlockSpec((1, h), lambda i: (0, 0)),
            pl.BlockSpec((h, h), lambda i: (0, 0)),
            pl.BlockSpec((h, 1), lambda i: (0, 0)),
            pl.BlockSpec((1, h), lambda i: (0, 0)),
            pl.BlockSpec((1, 1), lambda i: (0, 0)),
        ],
        out_specs=[
            pl.BlockSpec((rb, 1), lambda i: (i, 0)),
            pl.BlockSpec((rb, 1), lambda i: (i, 0)),
            pl.BlockSpec((1, 1), lambda i: (0, 0)),
        ],
        out_shape=[
            jax.ShapeDtypeStruct((n, 1), jnp.float32),
            jax.ShapeDtypeStruct((n, 1), jnp.float32),
            jax.ShapeDtypeStruct((1, 1), jnp.float32),
        ],
    )(aggp[:n], aggp[npad:npad + n], u1, d0, d1, b1.reshape(1, h), W2, lin_W,
      b2.reshape(1, h), lin_b.reshape(1, 1))

    zeros16g = jnp.zeros((16, _G), jnp.float32)
    buckets = _make_sc_pool(n, e)(srcR, dstR, qu.reshape(n), dis.reshape(n),
                                  batch, zeros16g)
    nw = _NC * _NS
    sums = jnp.sum(buckets[:16 * nw].reshape(nw * 16, _G), axis=0)
    cnt = jnp.sum(buckets[16 * nw:].reshape(nw * 16, _G), axis=0)
    return jnp.where(cnt > 0, sums / jnp.maximum(cnt, 1.0) + const[0, 0],
                     lin_b[0])
